# baseline (speedup 1.0000x reference)
import functools

import jax
import jax.numpy as jnp
from jax import lax
from jax.experimental import pallas as pl
from jax.experimental.pallas import tpu as pltpu
from jax.experimental.pallas import tpu_sc as plsc

B = 16384
D = 64
NC = 2
NS = 16
NW = NC * NS
BPW = B // NW
C = 64
NCHUNK = BPW // C


def _sc_gather(uid0, iid0, uid1, iid1, ue0, ie0, ue1, ie1):
    mesh = plsc.VectorSubcoreMesh(core_axis_name="c", subcore_axis_name="s")

    @functools.partial(
        pl.kernel,
        mesh=mesh,
        out_type=[jax.ShapeDtypeStruct((B, D), jnp.float32) for _ in range(4)],
        scratch_types=[
            pltpu.VMEM((BPW,), jnp.int32),
            pltpu.VMEM((BPW,), jnp.int32),
            pltpu.VMEM((C, 2 * D), jnp.float32),
            pltpu.VMEM((C, 2 * D), jnp.float32),
            pltpu.VMEM((BPW, D), jnp.float32),
            pltpu.SemaphoreType.DMA,
            pltpu.SemaphoreType.DMA,
        ],
        compiler_params=pltpu.CompilerParams(
            use_tc_tiling_on_sc=True, needs_layout_passes=False),
    )
    def k(uid0_h, iid0_h, uid1_h, iid1_h, ue0_h, ie0_h, ue1_h, ie1_h,
          u0_o, i0_o, u1_o, i1_o, pidx_v, half_v, buf_a, buf_b, rows_v,
          sem_a, sem_b):
        wid = lax.axis_index("s") * NC + lax.axis_index("c")
        base = wid * BPW
        lane16 = lax.iota(jnp.int32, 16)

        for ids_h, tab_h, out_h in (
            (uid0_h, ue0_h, u0_o),
            (iid0_h, ie0_h, i0_o),
            (uid1_h, ue1_h, u1_o),
            (iid1_h, ie1_h, i1_o),
        ):
            pltpu.sync_copy(ids_h.at[pl.ds(base, BPW)], pidx_v)

            def split_body(g):
                v = pidx_v[pl.ds(g * 16, 16)]
                half_v[pl.ds(g * 16, 16)] = (v & 1) * D
                pidx_v[pl.ds(g * 16, 16)] = v >> 1
            pl.loop(0, BPW // 16)(split_body)

            def fire(c, buf, s):
                pltpu.async_copy(tab_h.at[pidx_v.at[pl.ds(c * C, C)]], buf, s)

            def extract(c, buf, s):
                pltpu.make_async_copy(tab_h.at[pl.ds(0, C)], buf, s).wait()
                for g in range(C // 16):
                    rows = g * 16 + lane16
                    halves = half_v[pl.ds(c * C + g * 16, 16)]
                    for col in range(D):
                        colv = jnp.full((16,), col, jnp.int32) + halves
                        x = plsc.load_gather(buf, [rows, colv])
                        plsc.store_scatter(
                            rows_v,
                            [c * C + rows, jnp.full((16,), col, jnp.int32)],
                            x)

            fire(0, buf_a, sem_a)

            def chunk_pair(p):
                c0 = p * 2
                fire(c0 + 1, buf_b, sem_b)
                extract(c0, buf_a, sem_a)

                @pl.when(c0 + 2 < NCHUNK)
                def _():
                    fire(c0 + 2, buf_a, sem_a)
                extract(c0 + 1, buf_b, sem_b)
            pl.loop(0, NCHUNK // 2)(chunk_pair)

            pltpu.sync_copy(rows_v, out_h.at[pl.ds(base, BPW)])

    return k(uid0, iid0, uid1, iid1, ue0, ie0, ue1, ie1)


def _tc_body(u0_r, i0_r, u1_r, i1_r, s0_r, s1_r, o0_r, o1_r):
    p0 = jnp.dot(u0_r[...], s0_r[...], preferred_element_type=jnp.float32)
    o0_r[...] = jnp.sum(p0 * i0_r[...], axis=1, keepdims=True)
    p1 = jnp.dot(u1_r[...], s1_r[...], preferred_element_type=jnp.float32)
    o1_r[...] = jnp.sum(p1 * i1_r[...], axis=1, keepdims=True)


def _tc_dense(u0, i0, u1, i1, s_0, s_1):
    R = 2048
    nb = B // R
    row_spec = pl.BlockSpec((R, D), lambda i: (i, 0))
    s_spec = pl.BlockSpec((D, D), lambda i: (0, 0))
    out_spec = pl.BlockSpec((R, 1), lambda i: (i, 0))
    return pl.pallas_call(
        _tc_body,
        grid=(nb,),
        in_specs=[row_spec, row_spec, row_spec, row_spec, s_spec, s_spec],
        out_specs=[out_spec, out_spec],
        out_shape=[jax.ShapeDtypeStruct((B, 1), jnp.float32) for _ in range(2)],
    )(u0, i0, u1, i1, s_0, s_1)


def kernel(user_ids_0, item_ids_0, user_ids_1, item_ids_1,
           user_emb_0, user_emb_1, item_emb_0, item_emb_1,
           S0, St_0, St_1):
    u0, i0, u1, i1 = _sc_gather(
        user_ids_0, item_ids_0, user_ids_1, item_ids_1,
        user_emb_0.reshape(-1, 2 * D), item_emb_0.reshape(-1, 2 * D),
        user_emb_1.reshape(-1, 2 * D), item_emb_1.reshape(-1, 2 * D))
    s_0 = jnp.concatenate([S0, St_0], axis=1)
    s_1 = jnp.concatenate([S0, St_1], axis=1)
    o0, o1 = _tc_dense(u0, i0, u1, i1, s_0, s_1)
    return jnp.concatenate([o0.reshape(1, B), o1.reshape(1, B)], axis=0)
